# baseline (speedup 1.0000x reference)
import functools

import jax
import jax.numpy as jnp
from jax import lax
from jax.experimental import pallas as pl
from jax.experimental.pallas import tpu as pltpu
from jax.experimental.pallas import tpu_sc as plsc

NUM_ROWS = 1000000
BATCH = 16384
EMBED_DIM = 64
FUSED = 2 * EMBED_DIM
LANES = 16
CHUNKS = 4
TC_BLK = 25600
R_SC = 204800
TAIL = 999936
A_ROWS = TAIL - R_SC
PAD_W = 129


def _sc_geometry():
    try:
        info = plsc.get_sparse_core_info()
        return info.num_cores, info.num_subcores
    except Exception:
        return 2, 16


def _fuse_body(u_ref, i_ref, o_ref):
    o_ref[...] = jnp.concatenate([u_ref[...], i_ref[...]], axis=0).T


def _fuse_tables_tc(user_embed_t, item_embed_t):
    off = R_SC // TC_BLK
    return pl.pallas_call(
        _fuse_body,
        grid=(pl.cdiv(A_ROWS, TC_BLK),),
        in_specs=[
            pl.BlockSpec((EMBED_DIM, TC_BLK), lambda j: (0, j + off)),
            pl.BlockSpec((EMBED_DIM, TC_BLK), lambda j: (0, j + off)),
        ],
        out_specs=pl.BlockSpec((TC_BLK, FUSED), lambda j: (j, 0)),
        out_shape=jax.ShapeDtypeStruct((A_ROWS, FUSED), jnp.float32),
    )(user_embed_t, item_embed_t)


def _fuse_tail_body(u_ref, i_ref, o_ref):
    o_ref[...] = jnp.concatenate([u_ref[...], i_ref[...]], axis=0).T[
        : NUM_ROWS - TAIL]


def _fuse_tail_tc(user_embed_t, item_embed_t):
    jt = TAIL // 128
    return pl.pallas_call(
        _fuse_tail_body,
        grid=(1,),
        in_specs=[
            pl.BlockSpec((EMBED_DIM, 128), lambda j: (0, jt)),
            pl.BlockSpec((EMBED_DIM, 128), lambda j: (0, jt)),
        ],
        out_specs=pl.BlockSpec((NUM_ROWS - TAIL, FUSED), lambda j: (0, 0)),
        out_shape=jax.ShapeDtypeStruct((NUM_ROWS - TAIL, FUSED), jnp.float32),
    )(user_embed_t, item_embed_t)


def _sc_fuse_body(blocks_per_w, nc,
                  u_hbm, i_hbm,
                  out_hbm,
                  u_stage, i_stage, pad_v, out_stage):
    wid = lax.axis_index("s") * nc + lax.axis_index("c")
    lane_iota = lax.iota(jnp.int32, LANES)

    def block(b, carry):
        col0 = (wid * blocks_per_w + b) * 128
        pltpu.sync_copy(u_hbm.at[:, pl.ds(col0, 128)], u_stage)
        pltpu.sync_copy(i_hbm.at[:, pl.ds(col0, 128)], i_stage)

        def dim(d, c2):
            for g in range(8):
                idx = (g * LANES + lane_iota) * PAD_W + d
                plsc.store_scatter(pad_v, [idx],
                                   u_stage[d, pl.ds(g * LANES, LANES)])
                plsc.store_scatter(pad_v, [idx + EMBED_DIM],
                                   i_stage[d, pl.ds(g * LANES, LANES)])
            return c2

        lax.fori_loop(0, EMBED_DIM, dim, 0)

        def row(r, c2):
            for g in range(8):
                out_stage[r, pl.ds(g * LANES, LANES)] = (
                    pad_v[pl.ds(r * PAD_W + g * LANES, LANES)])
            return c2

        lax.fori_loop(0, 128, row, 0)

        pltpu.sync_copy(out_stage, out_hbm.at[pl.ds(col0, 128)])
        return carry

    lax.fori_loop(0, blocks_per_w, block, 0)


def _fuse_tables_sc(user_embed_t, item_embed_t, nc, ns):
    blocks_per_w = R_SC // 128 // (nc * ns)
    mesh = plsc.VectorSubcoreMesh(core_axis_name="c", subcore_axis_name="s",
                                  num_cores=nc)
    run = pl.kernel(
        functools.partial(_sc_fuse_body, blocks_per_w, nc),
        out_type=jax.ShapeDtypeStruct((R_SC, FUSED), jnp.float32),
        mesh=mesh,
        scratch_types=[
            pltpu.VMEM((EMBED_DIM, 128), jnp.float32),
            pltpu.VMEM((EMBED_DIM, 128), jnp.float32),
            pltpu.VMEM((128 * PAD_W,), jnp.float32),
            pltpu.VMEM((128, FUSED), jnp.float32),
        ],
        compiler_params=pltpu.CompilerParams(needs_layout_passes=False,
                                             use_tc_tiling_on_sc=True),
    )
    return run(user_embed_t, item_embed_t)


def _body(rows_per_w, nc,
          uid_hbm, iid_hbm, b_hbm, a_hbm, c_hbm, ubias_hbm, ibias_hbm,
          out_hbm,
          uid_v, iid_v, idx_v, urows_v, irows_v, uar_v, iar_v, ucr_v, icr_v,
          ub_v, ib_v, out_v, t_v,
          sem):
    wid = lax.axis_index("s") * nc + lax.axis_index("c")
    base = wid * rows_per_w
    chunk = rows_per_w // CHUNKS

    pltpu.sync_copy(uid_hbm.at[pl.ds(base, rows_per_w)],
                    uid_v.at[pl.ds(0, rows_per_w)])
    pltpu.sync_copy(iid_hbm.at[pl.ds(base, rows_per_w)],
                    iid_v.at[pl.ds(0, rows_per_w)])

    def mkidx(g, carry):
        p = g * LANES
        u16 = uid_v[pl.ds(p, LANES)]
        i16 = iid_v[pl.ds(p, LANES)]
        idx_v[pl.ds(p, LANES)] = jnp.clip(u16 - R_SC, 0, A_ROWS - 1)
        idx_v[pl.ds(rows_per_w + p, LANES)] = jnp.clip(
            u16 - TAIL, 0, NUM_ROWS - TAIL - 1)
        idx_v[pl.ds(2 * rows_per_w + p, LANES)] = jnp.clip(
            i16 - R_SC, 0, A_ROWS - 1)
        idx_v[pl.ds(3 * rows_per_w + p, LANES)] = jnp.clip(
            i16 - TAIL, 0, NUM_ROWS - TAIL - 1)
        uid_v[pl.ds(p, LANES)] = jnp.minimum(u16, R_SC - 1)
        iid_v[pl.ds(p, LANES)] = jnp.minimum(i16, R_SC - 1)
        uid_v[pl.ds(rows_per_w + p, LANES)] = u16
        iid_v[pl.ds(rows_per_w + p, LANES)] = i16
        return carry

    lax.fori_loop(0, rows_per_w // LANES, mkidx, 0)

    cp_ub = pltpu.async_copy(
        ubias_hbm.at[uid_v.at[pl.ds(rows_per_w, rows_per_w)]], ub_v, sem)
    cp_ib = pltpu.async_copy(
        ibias_hbm.at[iid_v.at[pl.ds(rows_per_w, rows_per_w)]], ib_v, sem)

    lane_iota = lax.iota(jnp.int32, LANES)

    for ci in range(CHUNKS):
        cbase = ci * chunk
        cps = [
            pltpu.async_copy(
                b_hbm.at[uid_v.at[pl.ds(cbase, chunk)]], urows_v, sem),
            pltpu.async_copy(
                b_hbm.at[iid_v.at[pl.ds(cbase, chunk)]], irows_v, sem),
            pltpu.async_copy(
                a_hbm.at[idx_v.at[pl.ds(cbase, chunk)]], uar_v, sem),
            pltpu.async_copy(
                a_hbm.at[idx_v.at[pl.ds(2 * rows_per_w + cbase, chunk)]],
                iar_v, sem),
            pltpu.async_copy(
                c_hbm.at[idx_v.at[pl.ds(rows_per_w + cbase, chunk)]],
                ucr_v, sem),
            pltpu.async_copy(
                c_hbm.at[idx_v.at[pl.ds(3 * rows_per_w + cbase, chunk)]],
                icr_v, sem),
        ]
        for cp in cps:
            cp.wait()

        def group(g, carry):
            base_r = g * LANES
            for r in range(LANES):
                row = cbase + base_r + r
                urep = plsc.load_gather(
                    uid_v, [jnp.full((LANES,), rows_per_w, jnp.int32) + row])
                irep = plsc.load_gather(
                    iid_v, [jnp.full((LANES,), rows_per_w, jnp.int32) + row])
                u_inb = urep < R_SC
                u_ina = urep < TAIL
                i_inb = irep < R_SC
                i_ina = irep < TAIL
                acc = None
                for c in range(EMBED_DIM // LANES):
                    uv = jnp.where(
                        u_inb,
                        urows_v[base_r + r, pl.ds(c * LANES, LANES)],
                        jnp.where(
                            u_ina,
                            uar_v[base_r + r, pl.ds(c * LANES, LANES)],
                            ucr_v[base_r + r, pl.ds(c * LANES, LANES)]))
                    iv = jnp.where(
                        i_inb,
                        irows_v[base_r + r,
                                pl.ds(EMBED_DIM + c * LANES, LANES)],
                        jnp.where(
                            i_ina,
                            iar_v[base_r + r,
                                  pl.ds(EMBED_DIM + c * LANES, LANES)],
                            icr_v[base_r + r,
                                  pl.ds(EMBED_DIM + c * LANES, LANES)]))
                    prod = uv * iv
                    acc = prod if acc is None else acc + prod
                t_v[pl.ds(r * (LANES + 1), LANES)] = acc
            row_base = lane_iota * (LANES + 1)
            out16 = plsc.load_gather(t_v, [row_base])
            for c in range(1, LANES):
                out16 = out16 + plsc.load_gather(t_v, [row_base + c])
            out_v[pl.ds(cbase + base_r, LANES)] = out16
            return carry

        lax.fori_loop(0, chunk // LANES, group, 0)

    cp_ub.wait()
    cp_ib.wait()

    def bias_group(g, carry):
        base_r = g * LANES
        out16 = (out_v[pl.ds(base_r, LANES)]
                 + ub_v[pl.ds(base_r, LANES)]
                 + ib_v[pl.ds(base_r, LANES)])
        out_v[pl.ds(base_r, LANES)] = out16
        return carry

    lax.fori_loop(0, rows_per_w // LANES, bias_group, 0)

    pltpu.sync_copy(out_v, out_hbm.at[pl.ds(base, rows_per_w)])


def kernel(user_ids, item_ids, user_embed, item_embed, user_bias, item_bias):
    nc, ns = _sc_geometry()
    nw = nc * ns
    rows_per_w = BATCH // nw

    u_t = user_embed.T
    i_t = item_embed.T
    fused_b = _fuse_tables_sc(u_t, i_t, nc, ns)
    fused_a = _fuse_tables_tc(u_t, i_t)
    fused_c = _fuse_tail_tc(u_t, i_t)

    mesh = plsc.VectorSubcoreMesh(core_axis_name="c", subcore_axis_name="s",
                                  num_cores=nc)

    run = pl.kernel(
        functools.partial(_body, rows_per_w, nc),
        out_type=jax.ShapeDtypeStruct((BATCH,), jnp.float32),
        mesh=mesh,
        scratch_types=[
            pltpu.VMEM((2 * rows_per_w,), jnp.int32),
            pltpu.VMEM((2 * rows_per_w,), jnp.int32),
            pltpu.VMEM((4 * rows_per_w,), jnp.int32),
            pltpu.VMEM((rows_per_w // CHUNKS, FUSED), jnp.float32),
            pltpu.VMEM((rows_per_w // CHUNKS, FUSED), jnp.float32),
            pltpu.VMEM((rows_per_w // CHUNKS, FUSED), jnp.float32),
            pltpu.VMEM((rows_per_w // CHUNKS, FUSED), jnp.float32),
            pltpu.VMEM((rows_per_w // CHUNKS, FUSED), jnp.float32),
            pltpu.VMEM((rows_per_w // CHUNKS, FUSED), jnp.float32),
            pltpu.VMEM((rows_per_w,), jnp.float32),
            pltpu.VMEM((rows_per_w,), jnp.float32),
            pltpu.VMEM((rows_per_w,), jnp.float32),
            pltpu.VMEM((LANES * (LANES + 1),), jnp.float32),
            pltpu.SemaphoreType.DMA,
        ],
        compiler_params=pltpu.CompilerParams(needs_layout_passes=False,
                                             use_tc_tiling_on_sc=False),
    )
    return run(user_ids, item_ids, fused_b, fused_a, fused_c,
               user_bias.reshape(-1), item_bias.reshape(-1))


# speedup vs baseline: 4.9482x; 4.9482x over previous
import functools

import jax
import jax.numpy as jnp
from jax import lax
from jax.experimental import pallas as pl
from jax.experimental.pallas import tpu as pltpu
from jax.experimental.pallas import tpu_sc as plsc

NUM_ROWS = 1000000
BATCH = 16384
EMBED_DIM = 64
FUSED = 2 * EMBED_DIM
LANES = 16
CHUNKS = 2
TC_BLK = 25600


def _sc_geometry():
    try:
        info = plsc.get_sparse_core_info()
        return info.num_cores, info.num_subcores
    except Exception:
        return 2, 16


def _fuse_body(u_ref, i_ref, o_ref):
    o_ref[...] = jnp.concatenate([u_ref[...], i_ref[...]], axis=0).T


def _fuse_tables(user_embed_t, item_embed_t):
    return pl.pallas_call(
        _fuse_body,
        grid=(pl.cdiv(NUM_ROWS, TC_BLK),),
        in_specs=[
            pl.BlockSpec((EMBED_DIM, TC_BLK), lambda j: (0, j)),
            pl.BlockSpec((EMBED_DIM, TC_BLK), lambda j: (0, j)),
        ],
        out_specs=pl.BlockSpec((TC_BLK, FUSED), lambda j: (j, 0)),
        out_shape=jax.ShapeDtypeStruct((NUM_ROWS, FUSED), jnp.float32),
    )(user_embed_t, item_embed_t)


def _body(rows_per_w, nc,
          uid_hbm, iid_hbm, emb_hbm, ubias_hbm, ibias_hbm,
          out_hbm,
          uid_v, iid_v, urows_v, irows_v, ub_v, ib_v, out_v, t_v,
          sem):
    wid = lax.axis_index("s") * nc + lax.axis_index("c")
    base = wid * rows_per_w
    chunk = rows_per_w // CHUNKS

    pltpu.sync_copy(uid_hbm.at[pl.ds(base, rows_per_w)], uid_v)
    pltpu.sync_copy(iid_hbm.at[pl.ds(base, rows_per_w)], iid_v)

    cp_ub = pltpu.async_copy(ubias_hbm.at[uid_v], ub_v, sem)
    cp_ib = pltpu.async_copy(ibias_hbm.at[iid_v], ib_v, sem)

    lane_iota = lax.iota(jnp.int32, LANES)

    for ci in range(CHUNKS):
        cbase = ci * chunk
        cp_u = pltpu.async_copy(
            emb_hbm.at[uid_v.at[pl.ds(cbase, chunk)]], urows_v, sem)
        cp_i = pltpu.async_copy(
            emb_hbm.at[iid_v.at[pl.ds(cbase, chunk)]], irows_v, sem)
        cp_u.wait()
        cp_i.wait()

        def group(g, carry):
            base_r = g * LANES
            for r in range(LANES):
                acc = (urows_v[base_r + r, pl.ds(0, LANES)]
                       * irows_v[base_r + r, pl.ds(EMBED_DIM, LANES)])
                for c in range(1, EMBED_DIM // LANES):
                    acc = acc + (
                        urows_v[base_r + r, pl.ds(c * LANES, LANES)]
                        * irows_v[base_r + r,
                                  pl.ds(EMBED_DIM + c * LANES, LANES)])
                t_v[pl.ds(r * (LANES + 1), LANES)] = acc
            row_base = lane_iota * (LANES + 1)
            out16 = plsc.load_gather(t_v, [row_base])
            for c in range(1, LANES):
                out16 = out16 + plsc.load_gather(t_v, [row_base + c])
            out_v[pl.ds(cbase + base_r, LANES)] = out16
            return carry

        lax.fori_loop(0, chunk // LANES, group, 0)

    cp_ub.wait()
    cp_ib.wait()

    def bias_group(g, carry):
        base_r = g * LANES
        out16 = (out_v[pl.ds(base_r, LANES)]
                 + ub_v[pl.ds(base_r, LANES)]
                 + ib_v[pl.ds(base_r, LANES)])
        out_v[pl.ds(base_r, LANES)] = out16
        return carry

    lax.fori_loop(0, rows_per_w // LANES, bias_group, 0)

    pltpu.sync_copy(out_v, out_hbm.at[pl.ds(base, rows_per_w)])


def kernel(user_ids, item_ids, user_embed, item_embed, user_bias, item_bias):
    nc, ns = _sc_geometry()
    nw = nc * ns
    rows_per_w = BATCH // nw

    fused = _fuse_tables(user_embed.T, item_embed.T)

    mesh = plsc.VectorSubcoreMesh(core_axis_name="c", subcore_axis_name="s",
                                  num_cores=nc)

    run = pl.kernel(
        functools.partial(_body, rows_per_w, nc),
        out_type=jax.ShapeDtypeStruct((BATCH,), jnp.float32),
        mesh=mesh,
        scratch_types=[
            pltpu.VMEM((rows_per_w,), jnp.int32),
            pltpu.VMEM((rows_per_w,), jnp.int32),
            pltpu.VMEM((rows_per_w // CHUNKS, FUSED), jnp.float32),
            pltpu.VMEM((rows_per_w // CHUNKS, FUSED), jnp.float32),
            pltpu.VMEM((rows_per_w,), jnp.float32),
            pltpu.VMEM((rows_per_w,), jnp.float32),
            pltpu.VMEM((rows_per_w,), jnp.float32),
            pltpu.VMEM((LANES * (LANES + 1),), jnp.float32),
            pltpu.SemaphoreType.DMA,
        ],
        compiler_params=pltpu.CompilerParams(needs_layout_passes=False,
                                             use_tc_tiling_on_sc=False),
    )
    return run(user_ids, item_ids, fused,
               user_bias.reshape(-1), item_bias.reshape(-1))
